# Initial kernel scaffold; baseline (speedup 1.0000x reference)
import jax
import jax.numpy as jnp
from jax.experimental import pallas as pl


def kernel(x, embed):
    raise NotImplementedError("write your pallas kernel here")


# speedup vs baseline: 1.1546x; 1.1546x over previous
import functools

import jax
import jax.numpy as jnp
from jax import lax
from jax.experimental import pallas as pl
from jax.experimental.pallas import tpu as pltpu
from jax.experimental.pallas import tpu_sc as plsc

N = 16384
K = 8192
D = 256
TN = 512
GRID = N // TN


_CHUNK = 2736
_NEG_INF = float("-inf")


def _round_f32_to_bf16_rte(v):
    u = jax.lax.bitcast_convert_type(v, jnp.uint32)
    r = (u + jnp.uint32(0x7FFF) + ((u >> jnp.uint32(16)) & jnp.uint32(1))) \
        & jnp.uint32(0xFFFF0000)
    return jax.lax.bitcast_convert_type(r, jnp.float32)


def _dist_argmax_kernel(x_ref, et_ref, a_ref, c_ref, ind_ref):
    x = x_ref[...]
    et = et_ref[...]
    b2 = 2.0 * jnp.dot(x, et, preferred_element_type=jnp.float32)
    dist = -((a_ref[...] - b2) + c_ref[...])
    kiota = lax.broadcasted_iota(jnp.int32, (TN, K), 1)
    chunk_of_k = ((kiota >= _CHUNK).astype(jnp.int32)
                  + (kiota >= 2 * _CHUNK).astype(jnp.int32))
    m0 = jnp.max(jnp.where(chunk_of_k == 0, dist, _NEG_INF), axis=1)
    m1 = jnp.max(jnp.where(chunk_of_k == 1, dist, _NEG_INF), axis=1)
    m2 = jnp.max(jnp.where(chunk_of_k == 2, dist, _NEG_INF), axis=1)
    acc = _round_f32_to_bf16_rte(m0)
    chsel = jnp.zeros(m0.shape, jnp.int32)
    win1 = m1 > acc
    acc = jnp.where(win1, _round_f32_to_bf16_rte(m1), acc)
    chsel = jnp.where(win1, 1, chsel)
    win2 = m2 > acc
    chsel = jnp.where(win2, 2, chsel)
    sel = chunk_of_k == chsel[:, None]
    ind_ref[0, 0, :] = jnp.argmax(jnp.where(sel, dist, _NEG_INF),
                                  axis=-1).astype(jnp.int32)


def _argmax_indices(x_flat, embed_t, a, c):
    return pl.pallas_call(
        _dist_argmax_kernel,
        grid=(GRID,),
        in_specs=[
            pl.BlockSpec((TN, D), lambda i: (i, 0)),
            pl.BlockSpec((D, K), lambda i: (0, 0)),
            pl.BlockSpec((TN, 1), lambda i: (i, 0)),
            pl.BlockSpec((1, K), lambda i: (0, 0)),
        ],
        out_specs=pl.BlockSpec((1, 1, TN), lambda i: (i, 0, 0)),
        out_shape=jax.ShapeDtypeStruct((GRID, 1, TN), jnp.int32),
    )(x_flat, embed_t, a, c)


_SC_INFO = plsc.get_sparse_core_info()
_NC = _SC_INFO.num_cores
_NS = _SC_INFO.num_subcores
_NW = _NC * _NS
_BPW = N // _NW
_CS = 128
_NCH = _BPW // _CS


_gather_mesh = plsc.VectorSubcoreMesh(core_axis_name="c", subcore_axis_name="s")


@functools.partial(
    pl.kernel,
    mesh=_gather_mesh,
    out_type=jax.ShapeDtypeStruct((N, D), jnp.float32),
    scratch_types=[
        pltpu.VMEM((_NCH, _CS), jnp.int32),
        pltpu.VMEM((_CS, D), jnp.float32),
        pltpu.VMEM((_CS, D), jnp.float32),
        pltpu.SemaphoreType.DMA,
        pltpu.SemaphoreType.DMA,
    ],
)
def _gather_sc(table_hbm, idx_hbm, out_hbm, idx_v, rows0, rows1, sem0, sem1):
    wid = lax.axis_index("s") * _NC + lax.axis_index("c")
    base = wid * _BPW
    pltpu.sync_copy(idx_hbm.at[wid], idx_v)
    bufs = (rows0, rows1)
    sems = (sem0, sem1)
    pltpu.async_copy(table_hbm.at[idx_v.at[0]], bufs[0], sems[0])
    for ch in range(_NCH):
        pltpu.make_async_copy(table_hbm.at[idx_v.at[ch]], bufs[ch % 2],
                              sems[ch % 2]).wait()
        if ch + 1 < _NCH:
            pltpu.async_copy(table_hbm.at[idx_v.at[ch + 1]],
                             bufs[(ch + 1) % 2], sems[(ch + 1) % 2])
        pltpu.sync_copy(bufs[ch % 2], out_hbm.at[pl.ds(base + ch * _CS, _CS)])


def kernel(x, embed):
    shape = x.shape
    x_flat = x.reshape(-1, D)
    embed_t = embed.T
    a = jnp.sum(x_flat ** 2, axis=1, keepdims=True)
    c = jnp.sum(embed_t ** 2, axis=0, keepdims=True)
    ind = _argmax_indices(x_flat, embed_t, a, c)
    ind_flat = ind.reshape(N)
    quantized = _gather_sc(embed, ind_flat.reshape(_NW, _NCH, _CS))
    return quantized.reshape(shape), ind_flat.reshape(shape[:-1])
